# Initial kernel scaffold; baseline (speedup 1.0000x reference)
import jax
import jax.numpy as jnp
from jax.experimental import pallas as pl


def kernel(sparse_idx, varlen_idx, varlen_len, dense, emb_table, out_bias):
    raise NotImplementedError("write your pallas kernel here")


# speedup vs baseline: 4.1147x; 4.1147x over previous
import functools

import jax
import jax.numpy as jnp
from jax import lax
from jax.experimental import pallas as pl
from jax.experimental.pallas import tpu as pltpu
from jax.experimental.pallas import tpu_sc as plsc

B = 4096
N_SPARSE = 26
HIST = 50
N_DENSE = 13
VOCAB = 100000
DIM = 64
OUT_W = N_SPARSE * DIM + DIM + N_DENSE

_info = plsc.get_sparse_core_info()
NC, NS, L = _info.num_cores, _info.num_subcores, _info.num_lanes
NW = NC * NS
RPW = B // NW
C = 16
NCHUNK = RPW // C

SP_PER_CHUNK = C * N_SPARSE
VL_PER_CHUNK = C * HIST
DN_PER_CHUNK = C * N_DENSE
SP_SLICE = 104
VL_SLICE = 80


def _sc_body(spi_hbm, vli_hbm, len_hbm, dn_hbm, bias_hbm, table_hbm, out_hbm,
             spidx, vlidx, lenb, denb, biasb, gsp, gvl, outb, sem):
    wid = lax.axis_index("s") * NC + lax.axis_index("c")
    wbase = wid * RPW

    pltpu.sync_copy(bias_hbm, biasb)
    biasv = biasb[...]

    for chunk in range(NCHUNK):
        b0 = wbase + chunk * C

        pltpu.sync_copy(spi_hbm.at[pl.ds(b0 * N_SPARSE, SP_PER_CHUNK)], spidx)
        pltpu.sync_copy(vli_hbm.at[pl.ds(b0 * HIST, VL_PER_CHUNK)], vlidx)
        pltpu.sync_copy(len_hbm.at[pl.ds(b0, C)], lenb)
        pltpu.sync_copy(dn_hbm.at[pl.ds(b0 * N_DENSE, DN_PER_CHUNK)],
                        denb.at[pl.ds(0, DN_PER_CHUNK)])

        copies = []
        for k in range(SP_PER_CHUNK // SP_SLICE):
            o = k * SP_SLICE
            copies.append(pltpu.async_copy(
                table_hbm.at[spidx.at[pl.ds(o, SP_SLICE)]],
                gsp.at[pl.ds(o, SP_SLICE)], sem))
        for k in range(VL_PER_CHUNK // VL_SLICE):
            o = k * VL_SLICE
            copies.append(pltpu.async_copy(
                table_hbm.at[vlidx.at[pl.ds(o, VL_SLICE)]],
                gvl.at[pl.ds(o, VL_SLICE)], sem))
        for cp in copies:
            cp.wait()

        lanes = lax.iota(jnp.int32, L)

        def sp_body(i, carry):
            r = i // (DIM // L)
            cs = (i % (DIM // L)) * L
            v = gsp[r, pl.ds(cs, L)]
            rowc = i // (N_SPARSE * DIM // L)
            w = (i % (N_SPARSE * DIM // L)) * L
            idxv = rowc * OUT_W + w + lanes
            plsc.store_scatter(outb, [idxv], v + biasv)
            return carry
        lax.fori_loop(0, C * N_SPARSE * DIM // L, sp_body, 0)

        def pool_body(c, carry):
            lenv = plsc.load_gather(lenb, [jnp.zeros((L,), jnp.int32) + c])
            lenf = lenv.astype(jnp.float32)
            inv = 1.0 / jnp.maximum(lenf, 1.0)
            zero = jnp.zeros((L,), jnp.float32)
            base = c * HIST

            def j_body(j, acc):
                m = lenv > j
                r = base + j
                return tuple(
                    acc[d] + jnp.where(m, gvl[r, pl.ds(d * L, L)], zero)
                    for d in range(DIM // L))
            acc = lax.fori_loop(0, HIST, j_body, (zero,) * (DIM // L))
            obase = c * OUT_W + N_SPARSE * DIM
            for d in range(DIM // L):
                plsc.store_scatter(outb, [obase + d * L + lanes],
                                   acc[d] * inv + biasv)

            didx = c * N_DENSE + lanes
            dv = plsc.load_gather(denb, [didx])
            plsc.store_scatter(outb, [obase + DIM + lanes], dv + biasv,
                               mask=lanes < N_DENSE)
            return carry
        lax.fori_loop(0, C, pool_body, 0)

        pltpu.sync_copy(outb, out_hbm.at[pl.ds(b0 * OUT_W, C * OUT_W)])


@jax.jit
def _run(spi, vli, lens, dn, bias16, table):
    mesh = plsc.VectorSubcoreMesh(core_axis_name="c", subcore_axis_name="s")
    k = functools.partial(
        pl.kernel,
        out_type=jax.ShapeDtypeStruct((B * OUT_W,), jnp.float32),
        mesh=mesh,
        compiler_params=pltpu.CompilerParams(use_tc_tiling_on_sc=False,
                                             needs_layout_passes=False),
        scratch_types=[
            pltpu.VMEM((SP_PER_CHUNK,), jnp.int32),
            pltpu.VMEM((VL_PER_CHUNK,), jnp.int32),
            pltpu.VMEM((C,), jnp.int32),
            pltpu.VMEM((DN_PER_CHUNK + L,), jnp.float32),
            pltpu.VMEM((L,), jnp.float32),
            pltpu.VMEM((SP_PER_CHUNK, DIM), jnp.float32),
            pltpu.VMEM((VL_PER_CHUNK, DIM), jnp.float32),
            pltpu.VMEM((C * OUT_W,), jnp.float32),
            pltpu.SemaphoreType.DMA,
        ],
    )(_sc_body)
    return k(spi, vli, lens, dn, bias16, table).reshape(B, OUT_W)


def kernel(sparse_idx, varlen_idx, varlen_len, dense, emb_table, out_bias):
    spi = sparse_idx.astype(jnp.int32).reshape(-1)
    vli = varlen_idx.astype(jnp.int32).reshape(-1)
    lens = varlen_len.astype(jnp.int32)
    dn = dense.astype(jnp.float32).reshape(-1)
    bias16 = jnp.zeros((L,), jnp.float32) + out_bias.astype(jnp.float32)
    return _run(spi, vli, lens, dn, bias16, emb_table.astype(jnp.float32))


# speedup vs baseline: 4.3602x; 1.0597x over previous
import functools

import jax
import jax.numpy as jnp
from jax import lax
from jax.experimental import pallas as pl
from jax.experimental.pallas import tpu as pltpu
from jax.experimental.pallas import tpu_sc as plsc

B = 4096
N_SPARSE = 26
HIST = 50
N_DENSE = 13
VOCAB = 100000
DIM = 64
OUT_W = N_SPARSE * DIM + DIM + N_DENSE

_info = plsc.get_sparse_core_info()
NC, NS, L = _info.num_cores, _info.num_subcores, _info.num_lanes
NW = NC * NS
RPW = B // NW
C = 8
NCHUNK = RPW // C

SP_N = C * N_SPARSE
VL_N = C * HIST
DN_N = C * N_DENSE
SP_SLICE = 104
VL_SLICE = 80


def _sc_body(spi_hbm, vli_hbm, len_hbm, dn_hbm, bias_hbm, table_hbm, out_hbm,
             spidxA, vlidxA, lenbA, denbA, gspA, gvlA, outbA,
             spidxB, vlidxB, lenbB, denbB, gspB, gvlB, outbB,
             biasb, semgA, semgB, semoA, semoB):
    wid = lax.axis_index("s") * NC + lax.axis_index("c")
    wbase = wid * RPW

    sets = (
        (spidxA, vlidxA, lenbA, denbA, gspA, gvlA, outbA, semgA, semoA),
        (spidxB, vlidxB, lenbB, denbB, gspB, gvlB, outbB, semgB, semoB),
    )

    pltpu.sync_copy(bias_hbm, biasb)
    biasv = biasb[...]
    lanes = lax.iota(jnp.int32, L)

    def stage_in(k, s):
        spidx, vlidx, lenb, denb, gsp, gvl, _, semg, _ = sets[s]
        b0 = wbase + k * C
        pltpu.sync_copy(spi_hbm.at[pl.ds(b0 * N_SPARSE, SP_N)], spidx)
        pltpu.sync_copy(vli_hbm.at[pl.ds(b0 * HIST, VL_N)], vlidx)
        pltpu.sync_copy(len_hbm.at[pl.ds(b0, C)], lenb)
        pltpu.sync_copy(dn_hbm.at[pl.ds(b0 * N_DENSE, DN_N)],
                        denb.at[pl.ds(0, DN_N)])
        cps = []
        for j in range(SP_N // SP_SLICE):
            o = j * SP_SLICE
            cps.append(pltpu.async_copy(
                table_hbm.at[spidx.at[pl.ds(o, SP_SLICE)]],
                gsp.at[pl.ds(o, SP_SLICE)], semg))
        for j in range(VL_N // VL_SLICE):
            o = j * VL_SLICE
            cps.append(pltpu.async_copy(
                table_hbm.at[vlidx.at[pl.ds(o, VL_SLICE)]],
                gvl.at[pl.ds(o, VL_SLICE)], semg))
        return cps

    def compute(s):
        _, _, lenb, denb, gsp, gvl, outb, _, _ = sets[s]

        def row_body(c, carry):
            obase = c * OUT_W

            def f_body(f, carry2):
                row = c * N_SPARSE + f
                wb = obase + f * DIM
                for d in range(DIM // L):
                    v = gsp[row, pl.ds(d * L, L)]
                    plsc.store_scatter(outb, [wb + d * L + lanes], v + biasv)
                return carry2
            lax.fori_loop(0, N_SPARSE, f_body, 0)

            lenv = plsc.load_gather(lenb, [jnp.zeros((L,), jnp.int32) + c])
            inv = 1.0 / jnp.maximum(lenv.astype(jnp.float32), 1.0)
            zero = jnp.zeros((L,), jnp.float32)
            vbase = c * HIST

            def j_body(j, acc):
                m = lenv > j
                r = vbase + j
                return tuple(
                    acc[d] + jnp.where(m, gvl[r, pl.ds(d * L, L)], zero)
                    for d in range(DIM // L))
            acc = lax.fori_loop(0, HIST, j_body, (zero,) * (DIM // L))
            pbase = obase + N_SPARSE * DIM
            for d in range(DIM // L):
                plsc.store_scatter(outb, [pbase + d * L + lanes],
                                   acc[d] * inv + biasv)

            dmask = lanes < N_DENSE
            dv = plsc.load_gather(denb, [c * N_DENSE + lanes], mask=dmask)
            plsc.store_scatter(outb, [pbase + DIM + lanes], dv + biasv,
                               mask=dmask)
            return carry
        lax.fori_loop(0, C, row_body, 0)

    pend_g = [None, None]
    pend_o = [None, None]
    pend_g[0] = stage_in(0, 0)
    for k in range(NCHUNK):
        s = k % 2
        if k + 1 < NCHUNK:
            pend_g[1 - s] = stage_in(k + 1, 1 - s)
        for cp in pend_g[s]:
            cp.wait()
        if pend_o[s] is not None:
            pend_o[s].wait()
        compute(s)
        b0 = wbase + k * C
        pend_o[s] = pltpu.async_copy(
            sets[s][6], out_hbm.at[pl.ds(b0 * OUT_W, C * OUT_W)], sets[s][8])
    pend_o[0].wait()
    pend_o[1].wait()


@jax.jit
def _run(spi, vli, lens, dn, bias16, table):
    mesh = plsc.VectorSubcoreMesh(core_axis_name="c", subcore_axis_name="s")
    dbl = lambda: [
        pltpu.VMEM((SP_N,), jnp.int32),
        pltpu.VMEM((VL_N,), jnp.int32),
        pltpu.VMEM((C,), jnp.int32),
        pltpu.VMEM((DN_N + L,), jnp.float32),
        pltpu.VMEM((SP_N, DIM), jnp.float32),
        pltpu.VMEM((VL_N, DIM), jnp.float32),
        pltpu.VMEM((C * OUT_W,), jnp.float32),
    ]
    k = functools.partial(
        pl.kernel,
        out_type=jax.ShapeDtypeStruct((B * OUT_W,), jnp.float32),
        mesh=mesh,
        compiler_params=pltpu.CompilerParams(use_tc_tiling_on_sc=False,
                                             needs_layout_passes=False),
        scratch_types=dbl() + dbl() + [
            pltpu.VMEM((L,), jnp.float32),
            pltpu.SemaphoreType.DMA,
            pltpu.SemaphoreType.DMA,
            pltpu.SemaphoreType.DMA,
            pltpu.SemaphoreType.DMA,
        ],
    )(_sc_body)
    return k(spi, vli, lens, dn, bias16, table).reshape(B, OUT_W)


def kernel(sparse_idx, varlen_idx, varlen_len, dense, emb_table, out_bias):
    spi = sparse_idx.astype(jnp.int32).reshape(-1)
    vli = varlen_idx.astype(jnp.int32).reshape(-1)
    lens = varlen_len.astype(jnp.int32)
    dn = dense.astype(jnp.float32).reshape(-1)
    bias16 = jnp.zeros((L,), jnp.float32) + out_bias.astype(jnp.float32)
    return _run(spi, vli, lens, dn, bias16, emb_table.astype(jnp.float32))


# speedup vs baseline: 4.3855x; 1.0058x over previous
import functools

import jax
import jax.numpy as jnp
from jax import lax
from jax.experimental import pallas as pl
from jax.experimental.pallas import tpu as pltpu
from jax.experimental.pallas import tpu_sc as plsc

B = 4096
N_SPARSE = 26
HIST = 50
N_DENSE = 13
VOCAB = 100000
DIM = 64
OUT_W = N_SPARSE * DIM + DIM + N_DENSE

_info = plsc.get_sparse_core_info()
NC, NS, L = _info.num_cores, _info.num_subcores, _info.num_lanes
NW = NC * NS
RPW = B // NW
C = 16
NCHUNK = RPW // C

SP_N = C * N_SPARSE
VL_N = C * HIST
DN_N = C * N_DENSE
SP_SLICE = 104
VL_SLICE = 80


def _sc_body(spi_hbm, vli_hbm, len_hbm, dn_hbm, bias_hbm, table_hbm, out_hbm,
             spidx, vlidx, lenb, denb, biasb, gsp, gvl, outb, sem):
    wid = lax.axis_index("s") * NC + lax.axis_index("c")
    wbase = wid * RPW

    pltpu.sync_copy(bias_hbm, biasb)
    biasv = biasb[...]
    lanes = lax.iota(jnp.int32, L)

    for chunk in range(NCHUNK):
        b0 = wbase + chunk * C

        pltpu.sync_copy(spi_hbm.at[pl.ds(b0 * N_SPARSE, SP_N)], spidx)
        pltpu.sync_copy(vli_hbm.at[pl.ds(b0 * HIST, VL_N)], vlidx)
        pltpu.sync_copy(len_hbm.at[pl.ds(b0, C)], lenb)
        pltpu.sync_copy(dn_hbm.at[pl.ds(b0 * N_DENSE, DN_N)],
                        denb.at[pl.ds(0, DN_N)])

        cps = []
        for j in range(SP_N // SP_SLICE):
            o = j * SP_SLICE
            cps.append(pltpu.async_copy(
                table_hbm.at[spidx.at[pl.ds(o, SP_SLICE)]],
                gsp.at[pl.ds(o, SP_SLICE)], sem))
        for j in range(VL_N // VL_SLICE):
            o = j * VL_SLICE
            cps.append(pltpu.async_copy(
                table_hbm.at[vlidx.at[pl.ds(o, VL_SLICE)]],
                gvl.at[pl.ds(o, VL_SLICE)], sem))
        for cp in cps:
            cp.wait()

        def row_body(c, carry):
            colv = jnp.zeros((L,), jnp.int32) + c

            def f_body(f, carry2):
                row = c * N_SPARSE + f
                rb = f * DIM
                for d in range(DIM // L):
                    v = gsp[row, pl.ds(d * L, L)]
                    plsc.store_scatter(outb, [rb + d * L + lanes, colv],
                                       v + biasv)
                return carry2
            lax.fori_loop(0, N_SPARSE, f_body, 0)

            lenv = plsc.load_gather(lenb, [colv])
            inv = 1.0 / jnp.maximum(lenv.astype(jnp.float32), 1.0)
            zero = jnp.zeros((L,), jnp.float32)
            vbase = c * HIST

            def j_body(j, acc):
                m = lenv > j
                r = vbase + j
                return tuple(
                    acc[d] + jnp.where(m, gvl[r, pl.ds(d * L, L)], zero)
                    for d in range(DIM // L))
            acc = lax.fori_loop(0, HIST, j_body, (zero,) * (DIM // L))
            pb = N_SPARSE * DIM
            for d in range(DIM // L):
                plsc.store_scatter(outb, [pb + d * L + lanes, colv],
                                   acc[d] * inv + biasv)

            dmask = lanes < N_DENSE
            dv = plsc.load_gather(denb, [c * N_DENSE + lanes], mask=dmask)
            plsc.store_scatter(outb, [pb + DIM + lanes, colv], dv + biasv,
                               mask=dmask)
            return carry
        lax.fori_loop(0, C, row_body, 0)

        pltpu.sync_copy(outb, out_hbm.at[:, pl.ds(b0, C)])


@jax.jit
def _run(spi, vli, lens, dn, bias16, table):
    mesh = plsc.VectorSubcoreMesh(core_axis_name="c", subcore_axis_name="s")
    k = functools.partial(
        pl.kernel,
        out_type=jax.ShapeDtypeStruct((OUT_W, B), jnp.float32),
        mesh=mesh,
        compiler_params=pltpu.CompilerParams(use_tc_tiling_on_sc=False,
                                             needs_layout_passes=False),
        scratch_types=[
            pltpu.VMEM((SP_N,), jnp.int32),
            pltpu.VMEM((VL_N,), jnp.int32),
            pltpu.VMEM((C,), jnp.int32),
            pltpu.VMEM((DN_N + L,), jnp.float32),
            pltpu.VMEM((L,), jnp.float32),
            pltpu.VMEM((SP_N, DIM), jnp.float32),
            pltpu.VMEM((VL_N, DIM), jnp.float32),
            pltpu.VMEM((OUT_W, C), jnp.float32),
            pltpu.SemaphoreType.DMA,
        ],
    )(_sc_body)
    return k(spi, vli, lens, dn, bias16, table).T


def kernel(sparse_idx, varlen_idx, varlen_len, dense, emb_table, out_bias):
    spi = sparse_idx.astype(jnp.int32).reshape(-1)
    vli = varlen_idx.astype(jnp.int32).reshape(-1)
    lens = varlen_len.astype(jnp.int32)
    dn = dense.astype(jnp.float32).reshape(-1)
    bias16 = jnp.zeros((L,), jnp.float32) + out_bias.astype(jnp.float32)
    return _run(spi, vli, lens, dn, bias16, emb_table.astype(jnp.float32))


# speedup vs baseline: 5.1429x; 1.1727x over previous
import functools

import jax
import jax.numpy as jnp
from jax import lax
from jax.experimental import pallas as pl
from jax.experimental.pallas import tpu as pltpu
from jax.experimental.pallas import tpu_sc as plsc

B = 4096
N_SPARSE = 26
HIST = 50
N_DENSE = 13
VOCAB = 100000
DIM = 64
OUT_W = N_SPARSE * DIM + DIM + N_DENSE

_info = plsc.get_sparse_core_info()
NC, NS, L = _info.num_cores, _info.num_subcores, _info.num_lanes
NW = NC * NS
RPW = B // NW
C = 16
NCHUNK = RPW // C

SP_N = C * N_SPARSE
VL_N = C * HIST
SP_SLICE = 104
VL_SLICE = 80


def _sc_body(spi_hbm, vli_hbm, len_hbm, dn_hbm, bias_hbm, table_hbm, out_hbm,
             spidx, vlidx, lenb, denb, biasb, gsp, gvl, outb,
             semsp, semvl, semo):
    wid = lax.axis_index("s") * NC + lax.axis_index("c")
    wbase = wid * RPW

    pltpu.sync_copy(bias_hbm, biasb)
    pltpu.sync_copy(spi_hbm.at[pl.ds(wbase * N_SPARSE, RPW * N_SPARSE)], spidx)
    pltpu.sync_copy(vli_hbm.at[pl.ds(wbase * HIST, RPW * HIST)], vlidx)
    pltpu.sync_copy(len_hbm.at[pl.ds(wbase, RPW)], lenb)
    pltpu.sync_copy(dn_hbm.at[pl.ds(wbase * N_DENSE, RPW * N_DENSE)],
                    denb.at[pl.ds(0, RPW * N_DENSE)])
    biasv = biasb[...]
    lanes = lax.iota(jnp.int32, L)

    def fire_sp(k):
        o0 = k * SP_N
        return [pltpu.async_copy(
            table_hbm.at[spidx.at[pl.ds(o0 + j * SP_SLICE, SP_SLICE)]],
            gsp.at[pl.ds(j * SP_SLICE, SP_SLICE)], semsp)
            for j in range(SP_N // SP_SLICE)]

    def fire_vl(k):
        v0 = k * VL_N
        return [pltpu.async_copy(
            table_hbm.at[vlidx.at[pl.ds(v0 + j * VL_SLICE, VL_SLICE)]],
            gvl.at[pl.ds(j * VL_SLICE, VL_SLICE)], semvl)
            for j in range(VL_N // VL_SLICE)]

    pend_o = None
    sp_cp = fire_sp(0)
    vl_cp = fire_vl(0)
    for k in range(NCHUNK):
        brow = k * C

        for cp in sp_cp:
            cp.wait()
        if pend_o is not None:
            pend_o.wait()

        def row_body(c, carry):
            colv = jnp.zeros((L,), jnp.int32) + c

            def f_body(f, carry2):
                row = c * N_SPARSE + f
                rb = f * DIM
                for d in range(DIM // L):
                    v = gsp[row, pl.ds(d * L, L)]
                    plsc.store_scatter(outb, [rb + d * L + lanes, colv],
                                       v + biasv)
                return carry2
            lax.fori_loop(0, N_SPARSE, f_body, 0)
            return carry
        lax.fori_loop(0, C, row_body, 0)

        if k + 1 < NCHUNK:
            sp_next = fire_sp(k + 1)
        for cp in vl_cp:
            cp.wait()

        def pool_body(c, carry):
            colv = jnp.zeros((L,), jnp.int32) + c
            lenv = plsc.load_gather(lenb, [colv + brow])
            inv = 1.0 / jnp.maximum(lenv.astype(jnp.float32), 1.0)
            zero = jnp.zeros((L,), jnp.float32)
            vbase = c * HIST

            def j_body(j, acc):
                m = lenv > j
                r = vbase + j
                return tuple(
                    acc[d] + jnp.where(m, gvl[r, pl.ds(d * L, L)], zero)
                    for d in range(DIM // L))
            acc = lax.fori_loop(0, HIST, j_body, (zero,) * (DIM // L))
            pb = N_SPARSE * DIM
            for d in range(DIM // L):
                plsc.store_scatter(outb, [pb + d * L + lanes, colv],
                                   acc[d] * inv + biasv)

            dmask = lanes < N_DENSE
            dv = plsc.load_gather(denb, [(brow + c) * N_DENSE + lanes],
                                  mask=dmask)
            plsc.store_scatter(outb, [pb + DIM + lanes, colv], dv + biasv,
                               mask=dmask)
            return carry
        lax.fori_loop(0, C, pool_body, 0)

        if k + 1 < NCHUNK:
            sp_cp = sp_next
            vl_cp = fire_vl(k + 1)
        pend_o = pltpu.async_copy(outb, out_hbm.at[:, pl.ds(wbase + brow, C)],
                                  semo)
    pend_o.wait()


@jax.jit
def _run(spi, vli, lens, dn, bias16, table):
    mesh = plsc.VectorSubcoreMesh(core_axis_name="c", subcore_axis_name="s")
    k = functools.partial(
        pl.kernel,
        out_type=jax.ShapeDtypeStruct((OUT_W, B), jnp.float32),
        mesh=mesh,
        compiler_params=pltpu.CompilerParams(use_tc_tiling_on_sc=False,
                                             needs_layout_passes=False),
        scratch_types=[
            pltpu.VMEM((RPW * N_SPARSE,), jnp.int32),
            pltpu.VMEM((RPW * HIST,), jnp.int32),
            pltpu.VMEM((RPW,), jnp.int32),
            pltpu.VMEM((RPW * N_DENSE + L,), jnp.float32),
            pltpu.VMEM((L,), jnp.float32),
            pltpu.VMEM((SP_N, DIM), jnp.float32),
            pltpu.VMEM((VL_N, DIM), jnp.float32),
            pltpu.VMEM((OUT_W, C), jnp.float32),
            pltpu.SemaphoreType.DMA,
            pltpu.SemaphoreType.DMA,
            pltpu.SemaphoreType.DMA,
        ],
    )(_sc_body)
    return k(spi, vli, lens, dn, bias16, table).T


def kernel(sparse_idx, varlen_idx, varlen_len, dense, emb_table, out_bias):
    spi = sparse_idx.astype(jnp.int32).reshape(-1)
    vli = varlen_idx.astype(jnp.int32).reshape(-1)
    lens = varlen_len.astype(jnp.int32)
    dn = dense.astype(jnp.float32).reshape(-1)
    bias16 = jnp.zeros((L,), jnp.float32) + out_bias.astype(jnp.float32)
    return _run(spi, vli, lens, dn, bias16, emb_table.astype(jnp.float32))
